# baseline (speedup 1.0000x reference)
import functools

import jax
import jax.numpy as jnp
from jax import lax
from jax.experimental import pallas as pl
from jax.experimental.pallas import tpu as pltpu
from jax.experimental.pallas import tpu_sc as plsc

D_MODEL = 128
LANES = 16
NUM_CORES = 2
NUM_SUBCORES = 16
NUM_WORKERS = NUM_CORES * NUM_SUBCORES
PACK = 4
ROW = PACK * D_MODEL
CHUNK = 32
NBUF = 4


@functools.cache
def _build(n_quads, n_status):
    assert n_status ** PACK == 256
    assert n_quads % (NUM_WORKERS * CHUNK) == 0
    quads_per_w = n_quads // NUM_WORKERS
    n_chunks = quads_per_w // CHUNK
    assert n_chunks > 2 * NBUF
    combos_per_sub = n_status ** PACK // NUM_SUBCORES

    mesh = plsc.VectorSubcoreMesh(core_axis_name="c", subcore_axis_name="s")

    @functools.partial(
        pl.kernel,
        mesh=mesh,
        out_type=jax.ShapeDtypeStruct((n_quads, PACK, D_MODEL), jnp.float32),
        scratch_types=[
            pltpu.VMEM((n_chunks, CHUNK), jnp.int32),
            pltpu.VMEM((n_status, D_MODEL), jnp.float32),
            pltpu.VMEM((combos_per_sub, PACK, D_MODEL), jnp.float32),
            pltpu.VMEM_SHARED((n_status ** PACK, PACK, D_MODEL), jnp.float32),
            pltpu.VMEM((NBUF, CHUNK, PACK, D_MODEL), jnp.float32),
            pltpu.SemaphoreType.DMA,
            pltpu.SemaphoreType.DMA,
        ],
    )
    def lookup(ids_hbm, table_hbm, out_hbm, idx_v, table_v, build_v,
               combo_sh, rows_v, gsem, osem):
        sid = lax.axis_index("s")
        wid = sid * NUM_CORES + lax.axis_index("c")
        quad0 = wid * quads_per_w

        pltpu.sync_copy(ids_hbm.at[pl.ds(wid * n_chunks, n_chunks)], idx_v)
        pltpu.sync_copy(table_hbm, table_v)

        for cl in range(combos_per_sub):
            combo = sid * combos_per_sub + cl
            for p in range(PACK):
                part = lax.shift_right_logical(
                    combo, 2 * (PACK - 1 - p)) & (n_status - 1)
                for k in range(D_MODEL // LANES):
                    build_v[cl, p, pl.ds(k * LANES, LANES)] = (
                        table_v[part, pl.ds(k * LANES, LANES)])
        pltpu.sync_copy(
            build_v,
            combo_sh.at[pl.ds(sid * combos_per_sub, combos_per_sub)])
        plsc.subcore_barrier()

        def start_gather(g, b):
            pltpu.async_copy(combo_sh.at[idx_v.at[g]], rows_v.at[b], gsem)

        def wait_gather(b):
            pltpu.make_async_copy(
                combo_sh.at[idx_v.at[0]], rows_v.at[b], gsem).wait()

        def start_out(g, b):
            pltpu.async_copy(
                rows_v.at[b], out_hbm.at[pl.ds(quad0 + g * CHUNK, CHUNK)],
                osem)

        def wait_out(b):
            pltpu.make_async_copy(
                rows_v.at[b], out_hbm.at[pl.ds(quad0, CHUNK)], osem).wait()

        for b in range(NBUF):
            start_gather(b, b)

        def step(g, b):
            wait_gather(b)
            start_out(g, b)
            wait_out(b)
            start_gather(g + NBUF, b)

        full = (n_chunks - NBUF) // NBUF
        rem = (n_chunks - NBUF) % NBUF

        def body(i, carry):
            for b in range(NBUF):
                step(i * NBUF + b, b)
            return carry

        lax.fori_loop(0, full, body, 0, unroll=False)
        for j in range(rem):
            step(full * NBUF + j, j)

        for j in range(NBUF):
            g = n_chunks - NBUF + j
            b = g % NBUF
            wait_gather(b)
            start_out(g, b)
        for j in range(NBUF):
            wait_out(j)

    return lookup


def kernel(status_ids, table):
    batch, max_nodes = status_ids.shape
    n_rows = batch * max_nodes
    n_quads = n_rows // PACK
    v = status_ids.astype(jnp.int32).reshape(n_quads, PACK)
    q = ((v[:, 0] * 4 + v[:, 1]) * 4 + v[:, 2]) * 4 + v[:, 3]
    q = q.reshape(NUM_WORKERS * (n_quads // (NUM_WORKERS * CHUNK)), CHUNK)
    out = _build(n_quads, table.shape[0])(q, table)
    return out.reshape(batch, max_nodes, table.shape[1])


# speedup vs baseline: 1.0958x; 1.0958x over previous
import functools

import jax
import jax.numpy as jnp
from jax import lax
from jax.experimental import pallas as pl
from jax.experimental.pallas import tpu as pltpu
from jax.experimental.pallas import tpu_sc as plsc

D_MODEL = 128
LANES = 16
NUM_CORES = 2
NUM_SUBCORES = 16
NUM_WORKERS = NUM_CORES * NUM_SUBCORES
CHUNK = 128
NBUF = 4


@functools.cache
def _build(n_rows, n_status):
    assert n_rows % (NUM_WORKERS * CHUNK) == 0
    rows_per_w = n_rows // NUM_WORKERS
    n_chunks = rows_per_w // CHUNK
    n_groups = n_chunks // NBUF
    assert n_groups > 2 and n_chunks % NBUF == 0

    mesh = plsc.VectorSubcoreMesh(core_axis_name="c", subcore_axis_name="s")

    @functools.partial(
        pl.kernel,
        mesh=mesh,
        out_type=jax.ShapeDtypeStruct((n_rows, D_MODEL), jnp.float32),
        scratch_types=[
            pltpu.VMEM((n_chunks, CHUNK), jnp.int32),
            pltpu.VMEM_SHARED((n_status, D_MODEL), jnp.float32),
            pltpu.VMEM((n_status, D_MODEL), jnp.float32),
            pltpu.VMEM((NBUF, CHUNK, D_MODEL), jnp.float32),
            pltpu.SemaphoreType.DMA,
            pltpu.SemaphoreType.DMA,
        ],
    )
    def lookup(ids_hbm, table_hbm, out_hbm, idx_v, table_sh, table_v, rows_v,
               gsem, osem):
        wid = lax.axis_index("s") * NUM_CORES + lax.axis_index("c")
        row0 = wid * rows_per_w

        pltpu.sync_copy(ids_hbm.at[pl.ds(wid * n_chunks, n_chunks)], idx_v)
        pltpu.sync_copy(table_hbm, table_v)

        @pl.when(lax.axis_index("s") == 0)
        def _():
            pltpu.sync_copy(table_hbm, table_sh)

        plsc.subcore_barrier()

        def start_gather(g, b):
            pltpu.async_copy(table_sh.at[idx_v.at[g]], rows_v.at[b], gsem)

        def wait_gather(b):
            pltpu.make_async_copy(
                table_sh.at[idx_v.at[0]], rows_v.at[b], gsem).wait()

        def start_out(g, b):
            pltpu.async_copy(
                rows_v.at[b], out_hbm.at[pl.ds(row0 + g * CHUNK, CHUNK)], osem)

        def wait_out(b):
            pltpu.make_async_copy(
                rows_v.at[b], out_hbm.at[pl.ds(row0, CHUNK)], osem).wait()

        def build(g, b):
            def group_body(rg, carry):
                r0 = rg * LANES
                idv = idx_v[g, pl.ds(r0, LANES)]
                for j in range(LANES):
                    sid = idv[j]
                    for k in range(D_MODEL // LANES):
                        rows_v[b, r0 + j, pl.ds(k * LANES, LANES)] = (
                            table_v[sid, pl.ds(k * LANES, LANES)])
                return carry
            lax.fori_loop(0, CHUNK // LANES, group_body, 0, unroll=False)

        for b in range(NBUF - 1):
            start_gather(b, b)

        def stream_step(g, b):
            wait_gather(b)
            start_out(g, b)
            wait_out(b)
            start_gather(g + NBUF, b)

        def valu_step(g):
            build(g, NBUF - 1)
            start_out(g, NBUF - 1)
            wait_out(NBUF - 1)

        def body(i, carry):
            for b in range(NBUF - 1):
                stream_step(i * NBUF + b, b)
            valu_step(i * NBUF + NBUF - 1)
            return carry

        lax.fori_loop(0, n_groups - 1, body, 0, unroll=False)

        for b in range(NBUF - 1):
            g = (n_groups - 1) * NBUF + b
            wait_gather(b)
            start_out(g, b)
            wait_out(b)
        valu_step(n_chunks - 1)

    return lookup


def kernel(status_ids, table):
    batch, max_nodes = status_ids.shape
    n_rows = batch * max_nodes
    ids_flat = status_ids.astype(jnp.int32).reshape(
        NUM_WORKERS * (n_rows // (NUM_WORKERS * CHUNK)), CHUNK)
    out = _build(n_rows, table.shape[0])(ids_flat, table)
    return out.reshape(batch, max_nodes, table.shape[1])


# speedup vs baseline: 2.0756x; 1.8941x over previous
import functools

import jax
import jax.numpy as jnp
from jax import lax
from jax.experimental import pallas as pl
from jax.experimental.pallas import tpu as pltpu
from jax.experimental.pallas import tpu_sc as plsc

D_MODEL = 128
NUM_CORES = 2
NUM_SUBCORES = 16
NUM_WORKERS = NUM_CORES * NUM_SUBCORES
CHUNK = 128
NBUF = 4


@functools.cache
def _build(n_rows, n_status):
    assert n_rows % (NUM_WORKERS * CHUNK) == 0
    rows_per_w = n_rows // NUM_WORKERS
    n_chunks = rows_per_w // CHUNK
    assert n_chunks > NBUF and (n_chunks - NBUF) % NBUF == 0

    mesh = plsc.VectorSubcoreMesh(core_axis_name="c", subcore_axis_name="s")

    @functools.partial(
        pl.kernel,
        mesh=mesh,
        out_type=jax.ShapeDtypeStruct((n_rows, D_MODEL), jnp.float32),
        scratch_types=[
            pltpu.VMEM((n_chunks, CHUNK), jnp.int32),
            pltpu.VMEM_SHARED((n_status, D_MODEL), jnp.float32),
            pltpu.VMEM((NBUF, CHUNK, D_MODEL), jnp.float32),
            pltpu.SemaphoreType.DMA,
            pltpu.SemaphoreType.DMA,
        ],
    )
    def lookup(ids_hbm, table_hbm, out_hbm, idx_v, table_sh, rows_v, gsem,
               osem):
        wid = lax.axis_index("s") * NUM_CORES + lax.axis_index("c")
        row0 = wid * rows_per_w

        pltpu.sync_copy(ids_hbm.at[pl.ds(wid * n_chunks, n_chunks)], idx_v)

        @pl.when(lax.axis_index("s") == 0)
        def _():
            pltpu.sync_copy(table_hbm, table_sh)

        plsc.subcore_barrier()

        def start_gather(g, b):
            pltpu.async_copy(table_sh.at[idx_v.at[g]], rows_v.at[b], gsem)

        def wait_gather(b):
            pltpu.make_async_copy(
                table_sh.at[idx_v.at[0]], rows_v.at[b], gsem).wait()

        def start_out(g, b):
            pltpu.async_copy(
                rows_v.at[b], out_hbm.at[pl.ds(row0 + g * CHUNK, CHUNK)], osem)

        def wait_out(b):
            pltpu.make_async_copy(
                rows_v.at[b], out_hbm.at[pl.ds(row0, CHUNK)], osem).wait()

        for b in range(NBUF):
            start_gather(b, b)

        def body(i, carry):
            for b in range(NBUF):
                g = i * NBUF + b
                wait_gather(b)
                start_out(g, b)
                wait_out(b)
                start_gather(g + NBUF, b)
            return carry

        lax.fori_loop(0, (n_chunks - NBUF) // NBUF, body, 0, unroll=False)

        for b in range(NBUF):
            wait_gather(b)
            start_out(n_chunks - NBUF + b, b)
        for b in range(NBUF):
            wait_out(b)

    return lookup


def kernel(status_ids, table):
    batch, max_nodes = status_ids.shape
    n_rows = batch * max_nodes
    ids_flat = status_ids.astype(jnp.int32).reshape(
        NUM_WORKERS * (n_rows // (NUM_WORKERS * CHUNK)), CHUNK)
    out = _build(n_rows, table.shape[0])(ids_flat, table)
    return out.reshape(batch, max_nodes, table.shape[1])
